# baseline (speedup 1.0000x reference)
import functools

import jax
import jax.numpy as jnp
from jax import lax
from jax.experimental import pallas as pl
from jax.experimental.pallas import tpu as pltpu
from jax.experimental.pallas import tpu_sc as plsc

B = 16384
CPW = 8
WPW = 3
HIDDEN = 128
TAGS = 19
TAGS_PAD = 32
VSTRIDE = 33
NC, NS, L = 2, 16, 16
NW = NC * NS
WORDS_PER_TILE = B // NW
GROUPS = WORDS_PER_TILE // L
WIDX_PER_TILE = WORDS_PER_TILE * WPW
WIDX_CHUNK = 128
WIDX_ROWS = WIDX_PER_TILE // WIDX_CHUNK
CIDX_PER_TILE = WORDS_PER_TILE * CPW
N_WORD = 30000
OUT_PER_TILE = WORDS_PER_TILE * TAGS
OUT_ROWS_PER_TILE = OUT_PER_TILE // 128


def _wproj_body(a_ref, b_ref, o_ref):
    o_ref[...] = jnp.dot(a_ref[...].astype(jnp.bfloat16),
                         b_ref[...].astype(jnp.bfloat16),
                         preferred_element_type=jnp.float32)


def _cproj_body(a_ref, b_ref, bias_ref, o_ref):
    o_ref[...] = (jnp.dot(a_ref[...], b_ref[...],
                          preferred_element_type=jnp.float32)
                  + bias_ref[...])


def _sc_body(cids_hbm, widx_hbm, wproj_hbm, cproj_hbm, out_hbm,
             cidx_v, widx_v, wtmp_v, wsum_v, cproj_v, out_v, sem):
    wid = lax.axis_index("s") * NC + lax.axis_index("c")
    base = wid * WORDS_PER_TILE

    pltpu.sync_copy(cids_hbm.at[pl.ds(base * CPW, CIDX_PER_TILE)], cidx_v)
    pltpu.sync_copy(widx_hbm.at[pl.ds(base * WPW, WIDX_PER_TILE)], widx_v)

    iota = lax.iota(jnp.int32, L)
    for c in range(WIDX_PER_TILE // L):
        pos = iota + c * L
        chunk = widx_v[pl.ds(c * L, L)]
        widx_v[pl.ds(c * L, L)] = chunk * 4 + lax.rem(pos, WPW)

    copies = [
        pltpu.async_copy(wproj_hbm.at[widx_v.at[pl.ds(r * WIDX_CHUNK,
                                                      WIDX_CHUNK)]],
                         wtmp_v.at[pl.ds(r * WIDX_CHUNK, WIDX_CHUNK)], sem)
        for r in range(WIDX_ROWS)
    ]
    pltpu.sync_copy(cproj_hbm, cproj_v)
    for cp in copies:
        cp.wait()

    def presum(i4, carry):
        for u in range(4):
            i = i4 * 4 + u
            for h in range(2):
                co = h * L
                s = (wtmp_v[i * WPW, pl.ds(co, L)]
                     + wtmp_v[i * WPW + 1, pl.ds(co, L)]
                     + wtmp_v[i * WPW + 2, pl.ds(co, L)])
                wsum_v[pl.ds(i * VSTRIDE + co, L)] = s
        return carry

    lax.fori_loop(0, WORDS_PER_TILE // 4, presum, 0)

    def group(g, carry):
        lanes = iota + g * L
        cid = [plsc.load_gather(cidx_v, [lanes * CPW + k]) * VSTRIDE
               for k in range(CPW)]
        wbase = lanes * VSTRIDE
        obase = lanes * TAGS
        for t in range(TAGS):
            vals = [plsc.load_gather(cproj_v, [cid[k] + t])
                    for k in range(CPW)]
            vals.append(plsc.load_gather(wsum_v, [wbase + t]))
            while len(vals) > 1:
                vals = [vals[i] + vals[i + 1] if i + 1 < len(vals) else vals[i]
                        for i in range(0, len(vals), 2)]
            flat = obase + t
            plsc.store_scatter(
                out_v,
                [lax.shift_right_logical(flat, 7),
                 lax.bitwise_and(flat, jnp.full((L,), 127, jnp.int32))],
                jnp.exp(vals[0]))
        return carry

    lax.fori_loop(0, GROUPS, group, 0)
    pltpu.sync_copy(out_v, out_hbm.at[pl.ds(wid * OUT_ROWS_PER_TILE,
                                            OUT_ROWS_PER_TILE)])


@functools.partial(
    pl.kernel,
    out_type=jax.ShapeDtypeStruct((B * TAGS // 128, 128), jnp.float32),
    mesh=plsc.VectorSubcoreMesh(core_axis_name="c", subcore_axis_name="s"),
    compiler_params=pltpu.CompilerParams(needs_layout_passes=False,
                                         use_tc_tiling_on_sc=False),
    scratch_types=[
        pltpu.VMEM((CIDX_PER_TILE,), jnp.int32),
        pltpu.VMEM((WIDX_PER_TILE,), jnp.int32),
        pltpu.VMEM((WIDX_PER_TILE, TAGS_PAD), jnp.float32),
        pltpu.VMEM((WORDS_PER_TILE * VSTRIDE,), jnp.float32),
        pltpu.VMEM((512 * VSTRIDE,), jnp.float32),
        pltpu.VMEM((OUT_ROWS_PER_TILE, 128), jnp.float32),
        pltpu.SemaphoreType.DMA,
    ],
)
def _sc_gather_sum_exp(cids_hbm, widx_hbm, wproj_hbm, cproj_hbm, out_hbm,
                       cidx_v, widx_v, wtmp_v, wsum_v, cproj_v, out_v, sem):
    _sc_body(cids_hbm, widx_hbm, wproj_hbm, cproj_hbm, out_hbm,
             cidx_v, widx_v, wtmp_v, wsum_v, cproj_v, out_v, sem)


def kernel(char_ids, char_offsets, word_ids, char_table, word_table, W, b):
    del char_offsets
    n_word = word_table.shape[0]

    Wt = jnp.pad(W.T, ((0, 0), (0, TAGS_PAD - TAGS)))
    Wc = jnp.pad(W.T[:HIDDEN], ((0, 0), (0, VSTRIDE - TAGS)))
    Wcat = jnp.pad(jnp.concatenate(
        [Wt[HIDDEN * (j + 1):HIDDEN * (j + 2)] for j in range(WPW)],
        axis=1), ((0, 0), (0, TAGS_PAD)))
    char_pad = jnp.pad(char_table, ((0, 512 - char_table.shape[0]), (0, 0)))
    bias = jnp.pad(b, (0, VSTRIDE - TAGS)).reshape(1, VSTRIDE) / CPW

    rows_blk = 2000
    wproj = pl.pallas_call(
        _wproj_body,
        grid=(n_word // rows_blk,),
        in_specs=[
            pl.BlockSpec((rows_blk, HIDDEN), lambda i: (i, 0)),
            pl.BlockSpec((HIDDEN, 128), lambda i: (0, 0)),
        ],
        out_specs=pl.BlockSpec((rows_blk, 128), lambda i: (i, 0)),
        out_shape=jax.ShapeDtypeStruct((n_word, 128), jnp.float32),
    )(word_table, Wcat)
    wproj = wproj.reshape(n_word * 4, TAGS_PAD)

    cproj = pl.pallas_call(
        _cproj_body,
        out_shape=jax.ShapeDtypeStruct((512, VSTRIDE), jnp.float32),
    )(char_pad, Wc, bias).reshape(-1)

    widx = word_ids.astype(jnp.int32).reshape(-1)
    out = _sc_gather_sum_exp(char_ids.astype(jnp.int32), widx, wproj, cproj)
    return out.reshape(B, TAGS)


# speedup vs baseline: 1.0371x; 1.0371x over previous
import functools

import jax
import jax.numpy as jnp
from jax import lax
from jax.experimental import pallas as pl
from jax.experimental.pallas import tpu as pltpu
from jax.experimental.pallas import tpu_sc as plsc

B = 16384
CPW = 8
WPW = 3
HIDDEN = 128
TAGS = 19
TAGS_PAD = 32
VSTRIDE = 33
NC, NS, L = 2, 16, 16
NW = NC * NS
WORDS_PER_TILE = B // NW
GROUPS = WORDS_PER_TILE // L
WIDX_PER_TILE = WORDS_PER_TILE * WPW
WIDX_CHUNK = 128
WIDX_ROWS = WIDX_PER_TILE // WIDX_CHUNK
CIDX_PER_TILE = WORDS_PER_TILE * CPW
N_WORD = 30000
OUT_PER_TILE = WORDS_PER_TILE * TAGS
OUT_ROWS_PER_TILE = OUT_PER_TILE // 128


def _wproj_body(a_ref, b_ref, o_ref):
    o_ref[...] = jnp.dot(a_ref[...].astype(jnp.bfloat16),
                         b_ref[...].astype(jnp.bfloat16),
                         preferred_element_type=jnp.float32)


def _cproj_body(a_ref, b_ref, bias_ref, o_ref):
    o_ref[...] = (jnp.dot(a_ref[...], b_ref[...],
                          preferred_element_type=jnp.float32)
                  + bias_ref[...])


def _sc_body(cids_hbm, widx_hbm, wproj_hbm, cproj_hbm, out_hbm,
             cidx_v, widx_v, wtmp_v, wsum_v, cproj_v, out_v, sem):
    wid = lax.axis_index("s") * NC + lax.axis_index("c")
    base = wid * WORDS_PER_TILE

    pltpu.sync_copy(cids_hbm.at[pl.ds(base * CPW, CIDX_PER_TILE)], cidx_v)
    pltpu.sync_copy(widx_hbm.at[pl.ds(base * WPW, WIDX_PER_TILE)], widx_v)

    iota = lax.iota(jnp.int32, L)
    for c in range(WIDX_PER_TILE // L):
        pos = iota + c * L
        chunk = widx_v[pl.ds(c * L, L)]
        widx_v[pl.ds(c * L, L)] = chunk * 4 + lax.rem(pos, WPW)

    copies = [
        pltpu.async_copy(wproj_hbm.at[widx_v.at[pl.ds(r * WIDX_CHUNK,
                                                      WIDX_CHUNK)]],
                         wtmp_v.at[pl.ds(r * WIDX_CHUNK, WIDX_CHUNK)], sem)
        for r in range(WIDX_ROWS)
    ]
    pltpu.sync_copy(cproj_hbm, cproj_v)
    for cp in copies:
        cp.wait()

    def presum(i4, carry):
        for u in range(4):
            i = i4 * 4 + u
            for h in range(2):
                co = h * L
                s = (wtmp_v[i * WPW, pl.ds(co, L)]
                     + wtmp_v[i * WPW + 1, pl.ds(co, L)]
                     + wtmp_v[i * WPW + 2, pl.ds(co, L)])
                wsum_v[pl.ds(i * VSTRIDE + co, L)] = s
        return carry

    lax.fori_loop(0, WORDS_PER_TILE // 4, presum, 0)

    def group(g, carry):
        lanes = iota + g * L
        cid = [plsc.load_gather(cidx_v, [lanes * CPW + k]) * VSTRIDE
               for k in range(CPW)]
        wbase = lanes * VSTRIDE
        for t in range(TAGS):
            vals = [plsc.load_gather(cproj_v, [cid[k] + t])
                    for k in range(CPW)]
            vals.append(plsc.load_gather(wsum_v, [wbase + t]))
            while len(vals) > 1:
                vals = [vals[i] + vals[i + 1] if i + 1 < len(vals) else vals[i]
                        for i in range(0, len(vals), 2)]
            plsc.store_scatter(out_v, [lanes, jnp.full((L,), t, jnp.int32)],
                               jnp.exp(vals[0]))
        return carry

    lax.fori_loop(0, GROUPS, group, 0)
    pltpu.sync_copy(out_v, out_hbm.at[pl.ds(base, WORDS_PER_TILE)])


@functools.partial(
    pl.kernel,
    out_type=jax.ShapeDtypeStruct((B, TAGS), jnp.float32),
    mesh=plsc.VectorSubcoreMesh(core_axis_name="c", subcore_axis_name="s"),
    compiler_params=pltpu.CompilerParams(needs_layout_passes=False,
                                         use_tc_tiling_on_sc=False),
    scratch_types=[
        pltpu.VMEM((CIDX_PER_TILE,), jnp.int32),
        pltpu.VMEM((WIDX_PER_TILE,), jnp.int32),
        pltpu.VMEM((WIDX_PER_TILE, TAGS_PAD), jnp.float32),
        pltpu.VMEM((WORDS_PER_TILE * VSTRIDE,), jnp.float32),
        pltpu.VMEM((512 * VSTRIDE,), jnp.float32),
        pltpu.VMEM((WORDS_PER_TILE, TAGS), jnp.float32),
        pltpu.SemaphoreType.DMA,
    ],
)
def _sc_gather_sum_exp(cids_hbm, widx_hbm, wproj_hbm, cproj_hbm, out_hbm,
                       cidx_v, widx_v, wtmp_v, wsum_v, cproj_v, out_v, sem):
    _sc_body(cids_hbm, widx_hbm, wproj_hbm, cproj_hbm, out_hbm,
             cidx_v, widx_v, wtmp_v, wsum_v, cproj_v, out_v, sem)


def kernel(char_ids, char_offsets, word_ids, char_table, word_table, W, b):
    del char_offsets
    n_word = word_table.shape[0]

    Wt = jnp.pad(W.T, ((0, 0), (0, TAGS_PAD - TAGS)))
    Wc = jnp.pad(W.T[:HIDDEN], ((0, 0), (0, VSTRIDE - TAGS)))
    Wcat = jnp.pad(jnp.concatenate(
        [Wt[HIDDEN * (j + 1):HIDDEN * (j + 2)] for j in range(WPW)],
        axis=1), ((0, 0), (0, TAGS_PAD)))
    char_pad = jnp.pad(char_table, ((0, 512 - char_table.shape[0]), (0, 0)))
    bias = jnp.pad(b, (0, VSTRIDE - TAGS)).reshape(1, VSTRIDE) / CPW

    rows_blk = 2000
    wproj = pl.pallas_call(
        _wproj_body,
        grid=(n_word // rows_blk,),
        in_specs=[
            pl.BlockSpec((rows_blk, HIDDEN), lambda i: (i, 0)),
            pl.BlockSpec((HIDDEN, 128), lambda i: (0, 0)),
        ],
        out_specs=pl.BlockSpec((rows_blk, 128), lambda i: (i, 0)),
        out_shape=jax.ShapeDtypeStruct((n_word, 128), jnp.float32),
    )(word_table, Wcat)
    wproj = wproj.reshape(n_word * 4, TAGS_PAD)

    cproj = pl.pallas_call(
        _cproj_body,
        out_shape=jax.ShapeDtypeStruct((512, VSTRIDE), jnp.float32),
    )(char_pad, Wc, bias).reshape(-1)

    widx = word_ids.astype(jnp.int32).reshape(-1)
    return _sc_gather_sum_exp(char_ids.astype(jnp.int32), widx, wproj, cproj)
